# baseline (speedup 1.0000x reference)
import functools

import jax
import jax.numpy as jnp
from jax import lax
from jax.experimental import pallas as pl
from jax.experimental.pallas import tpu as pltpu
from jax.experimental.pallas import tpu_sc as plsc

_NC = 2
_NS = 16
_NW = _NC * _NS

_CHUNK = 512
_KV = 32768
_PC = 512
_LO_N = 360448
_LANES = 16


def _tpack_block(t_ref, o_ref):
    y = t_ref[...].T
    o_ref[...] = jnp.concatenate([y, jnp.zeros_like(y)], axis=1)


def _tc_transpose_pack(table_t, lo_n):
    embed, vocab = table_t.shape
    hi_n = vocab - lo_n
    blk0 = lo_n // _KV
    return pl.pallas_call(
        _tpack_block,
        grid=(pl.cdiv(hi_n, _KV),),
        in_specs=[pl.BlockSpec((embed, _KV), lambda i: (0, blk0 + i))],
        out_specs=pl.BlockSpec((_KV, 2 * embed), lambda i: (i, 0)),
        out_shape=jax.ShapeDtypeStruct((hi_n, 2 * embed), jnp.float32),
    )(table_t)


def _sc_transpose_pack(table_t):
    embed = table_t.shape[0]
    cols_per_w = _LO_N // _NW
    n_chunks = cols_per_w // _PC
    d = 2 * embed
    mesh = plsc.VectorSubcoreMesh(core_axis_name="c", subcore_axis_name="s")

    @functools.partial(
        pl.kernel,
        mesh=mesh,
        out_type=jax.ShapeDtypeStruct((_LO_N * d,), jnp.float32),
        scratch_types=[
            pltpu.VMEM((embed, _PC), jnp.float32),
            pltpu.VMEM((_PC * d,), jnp.float32),
        ],
        compiler_params=pltpu.CompilerParams(needs_layout_passes=False),
    )
    def pack_kernel(table_hbm, out_hbm, tin, tout):
        wid = lax.axis_index("s") * _NC + lax.axis_index("c")
        col0 = wid * cols_per_w

        @pl.loop(0, n_chunks)
        def _(c):
            a = pl.multiple_of(col0 + c * _PC, 128)
            pltpu.sync_copy(table_hbm.at[:, pl.ds(a, _PC)], tin)

            @pl.loop(0, _PC)
            def _(r):
                idx1 = jnp.full((_LANES,), r, jnp.int32)
                for k in range(embed // _LANES):
                    idx0 = jnp.arange(_LANES, dtype=jnp.int32) + k * _LANES
                    v = plsc.load_gather(tin, [idx0, idx1])
                    tout.at[pl.ds(r * d + k * _LANES, _LANES)][...] = v

            off = pl.multiple_of((col0 + c * _PC) * d, 8)
            pltpu.sync_copy(tout, out_hbm.at[pl.ds(off, _PC * d)])

    return pack_kernel(table_t).reshape(_LO_N, d)


def _sc_gather(table_wide, idx, name):
    n, = idx.shape
    d = table_wide.shape[1]
    b_per_w = n // _NW
    n_chunks = b_per_w // _CHUNK
    mesh = plsc.VectorSubcoreMesh(core_axis_name="c", subcore_axis_name="s")

    @functools.partial(
        pl.kernel,
        mesh=mesh,
        out_type=jax.ShapeDtypeStruct((n, d), jnp.float32),
        scratch_types=[
            pltpu.VMEM((b_per_w,), jnp.int32),
            pltpu.VMEM((_CHUNK, d), jnp.float32),
            pltpu.SemaphoreType.DMA,
        ],
        name=name,
    )
    def gather_kernel(table_hbm, idx_hbm, out_hbm, idx_v, rows_v, sem):
        wid = lax.axis_index("s") * _NC + lax.axis_index("c")
        base = wid * b_per_w
        pltpu.sync_copy(idx_hbm.at[pl.ds(base, b_per_w)], idx_v)

        @pl.loop(0, n_chunks)
        def _(c):
            off = c * _CHUNK
            pltpu.async_copy(
                table_hbm.at[idx_v.at[pl.ds(off, _CHUNK)]], rows_v, sem
            ).wait()
            pltpu.sync_copy(rows_v, out_hbm.at[pl.ds(base + off, _CHUNK)])

    return gather_kernel(table_wide, idx)


def _mlp_block(lo0, lo1, lo2, lo3, lo4, hi0, hi1, hi2, hi3, hi4, sel_ref,
               w1_ref, b1_ref, w2_ref, b2_ref, o_ref):
    h = b1_ref[...]
    embed = w1_ref.shape[1]
    los = (lo0, lo1, lo2, lo3, lo4)
    his = (hi0, hi1, hi2, hi3, hi4)
    for w in range(len(los)):
        s = sel_ref[w][:, None]
        rw = jnp.where(s == 1, los[w][:, :embed], his[w][:, :embed])
        h = h + jnp.dot(rw, w1_ref[w], preferred_element_type=jnp.float32)
    h = jnp.maximum(h, 0.0)
    o_ref[...] = (
        jnp.dot(h, w2_ref[...], preferred_element_type=jnp.float32) + b2_ref[...]
    )


def _tc_mlp(rows_lo, rows_hi, sel, w1s, b1, w2, b2, batch, block_b=1024):
    window, embed, hidden = w1s.shape
    out_dim = w2.shape[1]
    nb = batch // block_b
    row_specs = [
        pl.BlockSpec((block_b, 2 * embed), functools.partial(
            lambda w, i: (w * nb + i, 0), w))
        for w in range(window)
    ]
    return pl.pallas_call(
        _mlp_block,
        grid=(nb,),
        in_specs=row_specs + row_specs + [
            pl.BlockSpec((window, block_b), lambda i: (0, i)),
            pl.BlockSpec((window, embed, hidden), lambda i: (0, 0, 0)),
            pl.BlockSpec((1, hidden), lambda i: (0, 0)),
            pl.BlockSpec((hidden, out_dim), lambda i: (0, 0)),
            pl.BlockSpec((1, out_dim), lambda i: (0, 0)),
        ],
        out_specs=pl.BlockSpec((block_b, out_dim), lambda i: (i, 0)),
        out_shape=jax.ShapeDtypeStruct((batch, out_dim), jnp.float32),
    )(*([rows_lo] * window), *([rows_hi] * window), sel, w1s, b1, w2, b2)


def kernel(word_indices, table, W1, b1, W2, b2):
    batch, window = word_indices.shape
    vocab, embed = table.shape

    table_t = table.T
    table_lo = _sc_transpose_pack(table_t)
    table_hi = _tc_transpose_pack(table_t, _LO_N)

    idx_wmajor = word_indices.T.reshape(-1).astype(jnp.int32)
    idx_lo = jnp.minimum(idx_wmajor, _LO_N - 1)
    idx_hi = jnp.maximum(idx_wmajor - _LO_N, 0)
    sel = (idx_wmajor < _LO_N).astype(jnp.int32).reshape(window, batch)

    rows_lo = _sc_gather(table_lo, idx_lo, "gather_lo")
    rows_hi = _sc_gather(table_hi, idx_hi, "gather_hi")

    w1s = W1.reshape(window, embed, -1)
    return _tc_mlp(rows_lo, rows_hi, sel, w1s, b1.reshape(1, -1), W2,
                   b2.reshape(1, -1), batch)


# speedup vs baseline: 12.2734x; 12.2734x over previous
import functools

import jax
import jax.numpy as jnp
from jax import lax
from jax.experimental import pallas as pl
from jax.experimental.pallas import tpu as pltpu
from jax.experimental.pallas import tpu_sc as plsc

_NC = 2
_NS = 16
_NW = _NC * _NS

_CHUNK = 512
_KV = 32768


def _tpack_block(t_ref, o_ref):
    embed = t_ref.shape[0]
    o_ref[:, :embed] = t_ref[...].T


def _tc_transpose_pack(table_t):
    embed, vocab = table_t.shape
    return pl.pallas_call(
        _tpack_block,
        grid=(pl.cdiv(vocab, _KV),),
        in_specs=[pl.BlockSpec((embed, _KV), lambda i: (0, i))],
        out_specs=pl.BlockSpec((_KV, 2 * embed), lambda i: (i, 0)),
        out_shape=jax.ShapeDtypeStruct((vocab, 2 * embed), jnp.float32),
    )(table_t)


def _sc_gather(table_wide, idx):
    n, = idx.shape
    d = table_wide.shape[1]
    b_per_w = n // _NW
    n_chunks = b_per_w // _CHUNK
    mesh = plsc.VectorSubcoreMesh(core_axis_name="c", subcore_axis_name="s")

    @functools.partial(
        pl.kernel,
        mesh=mesh,
        out_type=jax.ShapeDtypeStruct((n, d), jnp.float32),
        scratch_types=[
            pltpu.VMEM((b_per_w,), jnp.int32),
            pltpu.VMEM((_CHUNK, d), jnp.float32),
            pltpu.SemaphoreType.DMA,
        ],
    )
    def gather_kernel(table_hbm, idx_hbm, out_hbm, idx_v, rows_v, sem):
        wid = lax.axis_index("s") * _NC + lax.axis_index("c")
        base = wid * b_per_w
        pltpu.sync_copy(idx_hbm.at[pl.ds(base, b_per_w)], idx_v)

        @pl.loop(0, n_chunks)
        def _(c):
            off = c * _CHUNK
            pltpu.async_copy(
                table_hbm.at[idx_v.at[pl.ds(off, _CHUNK)]], rows_v, sem
            ).wait()
            pltpu.sync_copy(rows_v, out_hbm.at[pl.ds(base + off, _CHUNK)])

    return gather_kernel(table_wide, idx)


def _mlp_block(r0, r1, r2, r3, r4, w1_ref, b1_ref, w2_ref, b2_ref, o_ref):
    h = b1_ref[...]
    embed = w1_ref.shape[1]
    for w, rw in enumerate((r0, r1, r2, r3, r4)):
        h = h + jnp.dot(rw[:, :embed], w1_ref[w],
                        preferred_element_type=jnp.float32)
    h = jnp.maximum(h, 0.0)
    o_ref[...] = (
        jnp.dot(h, w2_ref[...], preferred_element_type=jnp.float32) + b2_ref[...]
    )


def _tc_mlp(rows, w1s, b1, w2, b2, batch, block_b=1024):
    window, embed, hidden = w1s.shape
    out_dim = w2.shape[1]
    nb = batch // block_b
    row_specs = [
        pl.BlockSpec((block_b, 2 * embed), functools.partial(
            lambda w, i: (w * nb + i, 0), w))
        for w in range(window)
    ]
    return pl.pallas_call(
        _mlp_block,
        grid=(nb,),
        in_specs=row_specs + [
            pl.BlockSpec((window, embed, hidden), lambda i: (0, 0, 0)),
            pl.BlockSpec((1, hidden), lambda i: (0, 0)),
            pl.BlockSpec((hidden, out_dim), lambda i: (0, 0)),
            pl.BlockSpec((1, out_dim), lambda i: (0, 0)),
        ],
        out_specs=pl.BlockSpec((block_b, out_dim), lambda i: (i, 0)),
        out_shape=jax.ShapeDtypeStruct((batch, out_dim), jnp.float32),
    )(*([rows] * window), w1s, b1, w2, b2)


def kernel(word_indices, table, W1, b1, W2, b2):
    batch, window = word_indices.shape
    vocab, embed = table.shape

    table_wide = _tc_transpose_pack(table.T)

    idx_wmajor = word_indices.T.reshape(-1).astype(jnp.int32)

    rows = _sc_gather(table_wide, idx_wmajor)

    w1s = W1.reshape(window, embed, -1)
    return _tc_mlp(rows, w1s, b1.reshape(1, -1), W2, b2.reshape(1, -1), batch)


# speedup vs baseline: 12.2925x; 1.0016x over previous
import functools

import jax
import jax.numpy as jnp
from jax import lax
from jax.experimental import pallas as pl
from jax.experimental.pallas import tpu as pltpu
from jax.experimental.pallas import tpu_sc as plsc

_NC = 2
_NS = 16
_NW = _NC * _NS

_CHUNK = 512
_KV = 32768


def _tpack_block(t_ref, o_ref):
    embed = t_ref.shape[0]
    o_ref[:, :embed] = t_ref[...].T


def _tc_transpose_pack(table_t):
    embed, vocab = table_t.shape
    return pl.pallas_call(
        _tpack_block,
        grid=(pl.cdiv(vocab, _KV),),
        in_specs=[pl.BlockSpec((embed, _KV), lambda i: (0, i))],
        out_specs=pl.BlockSpec((_KV, 2 * embed), lambda i: (i, 0)),
        out_shape=jax.ShapeDtypeStruct((vocab, 2 * embed), jnp.float32),
    )(table_t)


def _sc_gather(table_wide, idx):
    n, = idx.shape
    d = table_wide.shape[1]
    b_per_w = n // _NW
    n_chunks = b_per_w // _CHUNK
    mesh = plsc.VectorSubcoreMesh(core_axis_name="c", subcore_axis_name="s")

    @functools.partial(
        pl.kernel,
        mesh=mesh,
        out_type=jax.ShapeDtypeStruct((n, d), jnp.float32),
        scratch_types=[
            pltpu.VMEM((b_per_w,), jnp.int32),
            pltpu.VMEM((_CHUNK, d), jnp.float32),
            pltpu.SemaphoreType.DMA,
        ],
    )
    def gather_kernel(table_hbm, idx_hbm, out_hbm, idx_v, rows_v, sem):
        wid = lax.axis_index("s") * _NC + lax.axis_index("c")
        base = wid * b_per_w
        pltpu.sync_copy(idx_hbm.at[pl.ds(base, b_per_w)], idx_v)

        @pl.loop(0, n_chunks)
        def _(c):
            off = c * _CHUNK
            pltpu.async_copy(
                table_hbm.at[idx_v.at[pl.ds(off, _CHUNK)]], rows_v, sem
            ).wait()
            pltpu.sync_copy(rows_v, out_hbm.at[pl.ds(base + off, _CHUNK)])

    return gather_kernel(table_wide, idx)


def _mlp_block(r0, r1, r2, r3, r4, w1_ref, b1_ref, w2_ref, b2_ref, o_ref):
    h = b1_ref[...]
    embed = w1_ref.shape[1]
    for w, rw in enumerate((r0, r1, r2, r3, r4)):
        h = h + jnp.dot(rw[:, :embed].astype(jnp.bfloat16),
                        w1_ref[w].astype(jnp.bfloat16),
                        preferred_element_type=jnp.float32)
    h = jnp.maximum(h, 0.0)
    o_ref[...] = (
        jnp.dot(h, w2_ref[...], preferred_element_type=jnp.float32) + b2_ref[...]
    )


def _tc_mlp(rows, w1s, b1, w2, b2, batch, block_b=1024):
    window, embed, hidden = w1s.shape
    out_dim = w2.shape[1]
    nb = batch // block_b
    row_specs = [
        pl.BlockSpec((block_b, 2 * embed), functools.partial(
            lambda w, i: (w * nb + i, 0), w))
        for w in range(window)
    ]
    return pl.pallas_call(
        _mlp_block,
        grid=(nb,),
        in_specs=row_specs + [
            pl.BlockSpec((window, embed, hidden), lambda i: (0, 0, 0)),
            pl.BlockSpec((1, hidden), lambda i: (0, 0)),
            pl.BlockSpec((hidden, out_dim), lambda i: (0, 0)),
            pl.BlockSpec((1, out_dim), lambda i: (0, 0)),
        ],
        out_specs=pl.BlockSpec((block_b, out_dim), lambda i: (i, 0)),
        out_shape=jax.ShapeDtypeStruct((batch, out_dim), jnp.float32),
    )(*([rows] * window), w1s, b1, w2, b2)


def kernel(word_indices, table, W1, b1, W2, b2):
    batch, window = word_indices.shape
    vocab, embed = table.shape

    table_wide = _tc_transpose_pack(table.T)

    idx_wmajor = word_indices.T.reshape(-1).astype(jnp.int32)

    rows = _sc_gather(table_wide, idx_wmajor)

    w1s = W1.reshape(window, embed, -1)
    return _tc_mlp(rows, w1s, b1.reshape(1, -1), W2, b2.reshape(1, -1), batch)


# speedup vs baseline: 12.3928x; 1.0082x over previous
import functools

import jax
import jax.numpy as jnp
from jax import lax
from jax.experimental import pallas as pl
from jax.experimental.pallas import tpu as pltpu
from jax.experimental.pallas import tpu_sc as plsc

_NC = 2
_NS = 16
_NW = _NC * _NS

_CHUNK = 640
_KV = 32768


def _tpack_block(t_ref, o_ref):
    embed = t_ref.shape[0]
    o_ref[:, :embed] = t_ref[...].T


def _tc_transpose_pack(table_t):
    embed, vocab = table_t.shape
    return pl.pallas_call(
        _tpack_block,
        grid=(pl.cdiv(vocab, _KV),),
        in_specs=[pl.BlockSpec((embed, _KV), lambda i: (0, i))],
        out_specs=pl.BlockSpec((_KV, 2 * embed), lambda i: (i, 0)),
        out_shape=jax.ShapeDtypeStruct((vocab, 2 * embed), jnp.float32),
    )(table_t)


def _sc_gather(table_wide, idx):
    n, = idx.shape
    d = table_wide.shape[1]
    b_per_w = n // _NW
    n_chunks = b_per_w // _CHUNK
    mesh = plsc.VectorSubcoreMesh(core_axis_name="c", subcore_axis_name="s")

    @functools.partial(
        pl.kernel,
        mesh=mesh,
        out_type=jax.ShapeDtypeStruct((n, d), jnp.float32),
        scratch_types=[
            pltpu.VMEM((b_per_w,), jnp.int32),
            pltpu.VMEM((_CHUNK, d), jnp.float32),
            pltpu.SemaphoreType.DMA,
        ],
    )
    def gather_kernel(table_hbm, idx_hbm, out_hbm, idx_v, rows_v, sem):
        wid = lax.axis_index("s") * _NC + lax.axis_index("c")
        base = wid * b_per_w
        pltpu.sync_copy(idx_hbm.at[pl.ds(base, b_per_w)], idx_v)

        @pl.loop(0, n_chunks)
        def _(c):
            off = c * _CHUNK
            pltpu.async_copy(
                table_hbm.at[idx_v.at[pl.ds(off, _CHUNK)]], rows_v, sem
            ).wait()
            pltpu.sync_copy(rows_v, out_hbm.at[pl.ds(base + off, _CHUNK)])

    return gather_kernel(table_wide, idx)


def _mlp_block(r0, r1, r2, r3, r4, w1_ref, b1_ref, w2_ref, b2_ref, o_ref):
    h = b1_ref[...]
    embed = w1_ref.shape[1]
    for w, rw in enumerate((r0, r1, r2, r3, r4)):
        h = h + jnp.dot(rw[:, :embed], w1_ref[w],
                        preferred_element_type=jnp.float32)
    h = jnp.maximum(h, 0.0)
    o_ref[...] = (
        jnp.dot(h, w2_ref[...], preferred_element_type=jnp.float32) + b2_ref[...]
    )


def _tc_mlp(rows, w1s, b1, w2, b2, batch, block_b=2048):
    window, embed, hidden = w1s.shape
    out_dim = w2.shape[1]
    nb = batch // block_b
    row_specs = [
        pl.BlockSpec((block_b, 2 * embed), functools.partial(
            lambda w, i: (w * nb + i, 0), w))
        for w in range(window)
    ]
    return pl.pallas_call(
        _mlp_block,
        grid=(nb,),
        in_specs=row_specs + [
            pl.BlockSpec((window, embed, hidden), lambda i: (0, 0, 0)),
            pl.BlockSpec((1, hidden), lambda i: (0, 0)),
            pl.BlockSpec((hidden, out_dim), lambda i: (0, 0)),
            pl.BlockSpec((1, out_dim), lambda i: (0, 0)),
        ],
        out_specs=pl.BlockSpec((block_b, out_dim), lambda i: (i, 0)),
        out_shape=jax.ShapeDtypeStruct((batch, out_dim), jnp.float32),
    )(*([rows] * window), w1s, b1, w2, b2)


def kernel(word_indices, table, W1, b1, W2, b2):
    batch, window = word_indices.shape
    vocab, embed = table.shape

    table_wide = _tc_transpose_pack(table.T)

    idx_wmajor = word_indices.T.reshape(-1).astype(jnp.int32)

    rows = _sc_gather(table_wide, idx_wmajor)

    w1s = W1.reshape(window, embed, -1)
    return _tc_mlp(rows, w1s, b1.reshape(1, -1), W2, b2.reshape(1, -1), batch)
